# baseline (speedup 1.0000x reference)
import functools

import jax
import jax.numpy as jnp
from jax import lax
from jax.experimental import pallas as pl
from jax.experimental.pallas import tpu as pltpu
from jax.experimental.pallas import tpu_sc as plsc

MSIZE = 32
TPT = 16
OBUF_W = TPT * MSIZE + 1
NUM_WORKERS = 32


def kernel(parts, weight):
    icount, ocount = parts.shape
    count = weight.shape[0]
    msize = weight.shape[-1]
    assert msize == MSIZE and ocount % TPT == 0

    n_tasks = icount * (ocount // TPT)
    assert n_tasks % NUM_WORKERS == 0
    tasks_per_w = n_tasks // NUM_WORKERS
    assert tasks_per_w % 2 == 0
    j_count = ocount // TPT

    w2d = weight.reshape(count, msize * msize)
    parts_flat = parts.reshape(icount * ocount)

    mesh = plsc.VectorSubcoreMesh(core_axis_name="c", subcore_axis_name="s")

    @functools.partial(
        pl.kernel,
        mesh=mesh,
        out_type=jax.ShapeDtypeStruct((icount * msize, ocount * msize),
                                      jnp.float32),
        scratch_types=[
            pltpu.VMEM((tasks_per_w * TPT,), jnp.int32),
            pltpu.VMEM((TPT, msize * msize), jnp.float32),
            pltpu.VMEM((TPT, msize * msize), jnp.float32),
            pltpu.VMEM((msize * TPT * MSIZE,), jnp.float32),
            pltpu.VMEM((msize * TPT * MSIZE,), jnp.float32),
            pltpu.SemaphoreType.DMA,
            pltpu.SemaphoreType.DMA,
            pltpu.SemaphoreType.DMA,
            pltpu.SemaphoreType.DMA,
        ],
        compiler_params=pltpu.CompilerParams(needs_layout_passes=False),
    )
    def run(parts_hbm, w_hbm, out_hbm, idx_v, tiles0, tiles1,
            obuf0, obuf1, gsem0, gsem1, osem0, osem1):
        wid = lax.axis_index("s") * 2 + lax.axis_index("c")
        task0 = wid * tasks_per_w
        iota = lax.iota(jnp.int32, 16)
        iota_hi = iota + 16
        tiles = (tiles0, tiles1)
        obufs = (obuf0, obuf1)
        gsems = (gsem0, gsem1)
        osems = (osem0, osem1)

        pltpu.sync_copy(
            parts_hbm.at[pl.ds(task0 * TPT, tasks_per_w * TPT)], idx_v)

        def gather(local_t, buf, sem):
            pltpu.make_async_copy(
                w_hbm.at[idx_v.at[pl.ds(local_t * TPT, TPT)]],
                buf, sem).start()

        def gather_wait(buf, sem):
            pltpu.make_async_copy(w_hbm.at[idx_v.at[pl.ds(0, TPT)]],
                                  buf, sem).wait()

        addr_lo = iota * (TPT * MSIZE)
        addr_hi = (iota + 16) * (TPT * MSIZE)

        def transpose(tiles_v, obuf_v):
            def tile_body(k, carry):
                base_lo = addr_lo + k * MSIZE
                base_hi = addr_hi + k * MSIZE
                for c in range(0, MSIZE, 8):
                    vs = [(tiles_v[k, pl.ds((c + d) * MSIZE, 16)],
                           tiles_v[k, pl.ds((c + d) * MSIZE + 16, 16)])
                          for d in range(8)]
                    for d in range(8):
                        plsc.store_scatter(obuf_v, [base_lo + (c + d)],
                                           vs[d][0])
                        plsc.store_scatter(obuf_v, [base_hi + (c + d)],
                                           vs[d][1])
                return carry

            lax.fori_loop(0, TPT, tile_body, 0, unroll=False)

        def out_copy_rows(obuf_v, task, sem):
            i = task // j_count
            j = task % j_count
            return [pltpu.make_async_copy(
                obuf_v.at[pl.ds(r * TPT * MSIZE, TPT * MSIZE)],
                out_hbm.at[i * msize + r,
                           pl.ds(j * TPT * MSIZE, TPT * MSIZE)],
                sem) for r in range(MSIZE)]

        gather(0, tiles[0], gsems[0])

        def loop_body(t, carry):
            for b in range(2):
                local_t = 2 * t + b
                task = task0 + local_t

                @pl.when(local_t + 1 < tasks_per_w)
                def _():
                    gather(local_t + 1, tiles[1 - b], gsems[1 - b])

                gather_wait(tiles[b], gsems[b])

                @pl.when(local_t >= 2)
                def _():
                    for d in out_copy_rows(obufs[b], task - 2, osems[b]):
                        d.wait()

                transpose(tiles[b], obufs[b])
                for d in out_copy_rows(obufs[b], task, osems[b]):
                    d.start()
            return carry

        lax.fori_loop(0, tasks_per_w // 2, loop_body, 0, unroll=False)

        for d in out_copy_rows(obufs[0], task0 + tasks_per_w - 2, osems[0]):
            d.wait()
        for d in out_copy_rows(obufs[1], task0 + tasks_per_w - 1, osems[1]):
            d.wait()

    return run(parts_flat, w2d)


# speedup vs baseline: 1.5195x; 1.5195x over previous
import functools

import jax
import jax.numpy as jnp
from jax import lax
from jax.experimental import pallas as pl
from jax.experimental.pallas import tpu as pltpu
from jax.experimental.pallas import tpu_sc as plsc

MSIZE = 32
TPT = 16
OBUF_W = TPT * MSIZE + 1
NUM_WORKERS = 32


def kernel(parts, weight):
    icount, ocount = parts.shape
    count = weight.shape[0]
    msize = weight.shape[-1]
    assert msize == MSIZE and ocount % TPT == 0

    n_tasks = icount * (ocount // TPT)
    assert n_tasks % NUM_WORKERS == 0
    tasks_per_w = n_tasks // NUM_WORKERS
    assert tasks_per_w % 2 == 0
    j_count = ocount // TPT

    w2d = weight.reshape(count, msize * msize)
    parts_flat = parts.reshape(icount * ocount)

    mesh = plsc.VectorSubcoreMesh(core_axis_name="c", subcore_axis_name="s")

    @functools.partial(
        pl.kernel,
        mesh=mesh,
        out_type=jax.ShapeDtypeStruct((icount * msize, ocount * msize),
                                      jnp.float32),
        scratch_types=[
            pltpu.VMEM((tasks_per_w * TPT,), jnp.int32),
            pltpu.VMEM((TPT, msize * msize), jnp.float32),
            pltpu.VMEM((TPT, msize * msize), jnp.float32),
            pltpu.VMEM((msize * OBUF_W,), jnp.float32),
            pltpu.VMEM((msize, TPT * MSIZE), jnp.float32),
            pltpu.VMEM((msize, TPT * MSIZE), jnp.float32),
            pltpu.SemaphoreType.DMA,
            pltpu.SemaphoreType.DMA,
            pltpu.SemaphoreType.DMA,
            pltpu.SemaphoreType.DMA,
        ],
        compiler_params=pltpu.CompilerParams(needs_layout_passes=False),
    )
    def run(parts_hbm, w_hbm, out_hbm, idx_v, tiles0, tiles1, opad,
            obuf0, obuf1, gsem0, gsem1, osem0, osem1):
        wid = lax.axis_index("s") * 2 + lax.axis_index("c")
        task0 = wid * tasks_per_w
        iota = lax.iota(jnp.int32, 16)
        iota_hi = iota + 16
        tiles = (tiles0, tiles1)
        obufs = (obuf0, obuf1)
        gsems = (gsem0, gsem1)
        osems = (osem0, osem1)

        pltpu.sync_copy(
            parts_hbm.at[pl.ds(task0 * TPT, tasks_per_w * TPT)], idx_v)

        def gather(local_t, buf, sem):
            pltpu.make_async_copy(
                w_hbm.at[idx_v.at[pl.ds(local_t * TPT, TPT)]],
                buf, sem).start()

        def gather_wait(buf, sem):
            pltpu.make_async_copy(w_hbm.at[idx_v.at[pl.ds(0, TPT)]],
                                  buf, sem).wait()

        addr_lo = iota * OBUF_W
        addr_hi = (iota + 16) * OBUF_W

        def transpose(tiles_v):
            def tile_body(k, carry):
                base_lo = addr_lo + k * MSIZE
                base_hi = addr_hi + k * MSIZE
                for c in range(0, MSIZE, 4):
                    vs = [(tiles_v[k, pl.ds((c + d) * MSIZE, 16)],
                           tiles_v[k, pl.ds((c + d) * MSIZE + 16, 16)])
                          for d in range(4)]
                    for d in range(4):
                        plsc.store_scatter(opad, [base_lo + (c + d)],
                                           vs[d][0])
                        plsc.store_scatter(opad, [base_hi + (c + d)],
                                           vs[d][1])
                return carry

            lax.fori_loop(0, TPT, tile_body, 0, unroll=False)

        def repack(obuf_v):
            def row_body(r, carry):
                for k in range(TPT):
                    v0 = opad[pl.ds(r * OBUF_W + k * MSIZE, 16)]
                    v1 = opad[pl.ds(r * OBUF_W + k * MSIZE + 16, 16)]
                    obuf_v[r, pl.ds(k * MSIZE, 16)] = v0
                    obuf_v[r, pl.ds(k * MSIZE + 16, 16)] = v1
                return carry

            lax.fori_loop(0, msize, row_body, 0, unroll=False)

        def out_copy(obuf_v, task, sem):
            i = task // j_count
            j = task % j_count
            return pltpu.make_async_copy(
                obuf_v,
                out_hbm.at[pl.ds(i * msize, msize),
                           pl.ds(j * TPT * MSIZE, TPT * MSIZE)],
                sem)

        gather(0, tiles[0], gsems[0])

        def loop_body(t, carry):
            for b in range(2):
                local_t = 2 * t + b
                task = task0 + local_t

                @pl.when(local_t + 1 < tasks_per_w)
                def _():
                    gather(local_t + 1, tiles[1 - b], gsems[1 - b])

                gather_wait(tiles[b], gsems[b])

                @pl.when(local_t >= 2)
                def _():
                    out_copy(obufs[b], task - 2, osems[b]).wait()

                transpose(tiles[b])
                repack(obufs[b])
                out_copy(obufs[b], task, osems[b]).start()
            return carry

        lax.fori_loop(0, tasks_per_w // 2, loop_body, 0, unroll=False)

        out_copy(obufs[0], task0 + tasks_per_w - 2, osems[0]).wait()
        out_copy(obufs[1], task0 + tasks_per_w - 1, osems[1]).wait()

    return run(parts_flat, w2d)


# speedup vs baseline: 1.5204x; 1.0006x over previous
import functools

import jax
import jax.numpy as jnp
from jax import lax
from jax.experimental import pallas as pl
from jax.experimental.pallas import tpu as pltpu
from jax.experimental.pallas import tpu_sc as plsc

MSIZE = 32
TPT = 16
OBUF_W = TPT * MSIZE + 1
NUM_WORKERS = 32


def kernel(parts, weight):
    icount, ocount = parts.shape
    count = weight.shape[0]
    msize = weight.shape[-1]
    assert msize == MSIZE and ocount % TPT == 0

    n_tasks = icount * (ocount // TPT)
    assert n_tasks % NUM_WORKERS == 0
    tasks_per_w = n_tasks // NUM_WORKERS
    assert tasks_per_w % 2 == 0
    j_count = ocount // TPT

    w2d = weight.reshape(count, msize * msize)
    parts_flat = parts.reshape(icount * ocount)

    mesh = plsc.VectorSubcoreMesh(core_axis_name="c", subcore_axis_name="s")

    @functools.partial(
        pl.kernel,
        mesh=mesh,
        out_type=jax.ShapeDtypeStruct((icount * msize, ocount * msize),
                                      jnp.float32),
        scratch_types=[
            pltpu.VMEM((tasks_per_w * TPT,), jnp.int32),
            pltpu.VMEM((TPT, msize * msize), jnp.float32),
            pltpu.VMEM((TPT, msize * msize), jnp.float32),
            pltpu.VMEM((msize * OBUF_W,), jnp.float32),
            pltpu.VMEM((msize, TPT * MSIZE), jnp.float32),
            pltpu.VMEM((msize, TPT * MSIZE), jnp.float32),
            pltpu.SemaphoreType.DMA,
            pltpu.SemaphoreType.DMA,
            pltpu.SemaphoreType.DMA,
            pltpu.SemaphoreType.DMA,
        ],
        compiler_params=pltpu.CompilerParams(needs_layout_passes=False),
    )
    def run(parts_hbm, w_hbm, out_hbm, idx_v, tiles0, tiles1, opad,
            obuf0, obuf1, gsem0, gsem1, osem0, osem1):
        wid = lax.axis_index("s") * 2 + lax.axis_index("c")
        task0 = wid * tasks_per_w
        iota = lax.iota(jnp.int32, 16)
        tiles = (tiles0, tiles1)
        obufs = (obuf0, obuf1)
        gsems = (gsem0, gsem1)
        osems = (osem0, osem1)

        pltpu.sync_copy(
            parts_hbm.at[pl.ds(task0 * TPT, tasks_per_w * TPT)], idx_v)

        def gather(local_t, buf, sem):
            pltpu.make_async_copy(
                w_hbm.at[idx_v.at[pl.ds(local_t * TPT, TPT)]],
                buf, sem).start()

        def gather_wait(buf, sem):
            pltpu.make_async_copy(w_hbm.at[idx_v.at[pl.ds(0, TPT)]],
                                  buf, sem).wait()

        addr_lo = iota * OBUF_W
        addr_hi = (iota + 16) * OBUF_W

        def transpose(tiles_v):
            def tile_body(k, carry):
                base_lo = addr_lo + k * MSIZE
                base_hi = addr_hi + k * MSIZE
                for c in range(0, MSIZE, 4):
                    vs = [(tiles_v[k, pl.ds((c + d) * MSIZE, 16)],
                           tiles_v[k, pl.ds((c + d) * MSIZE + 16, 16)])
                          for d in range(4)]
                    for d in range(4):
                        plsc.store_scatter(opad, [base_lo + (c + d)],
                                           vs[d][0])
                        plsc.store_scatter(opad, [base_hi + (c + d)],
                                           vs[d][1])
                return carry

            lax.fori_loop(0, TPT, tile_body, 0, unroll=False)

        def repack(obuf_v):
            def row_body(r, carry):
                for k in range(TPT):
                    v0 = opad[pl.ds(r * OBUF_W + k * MSIZE, 16)]
                    v1 = opad[pl.ds(r * OBUF_W + k * MSIZE + 16, 16)]
                    obuf_v[r, pl.ds(k * MSIZE, 16)] = v0
                    obuf_v[r, pl.ds(k * MSIZE + 16, 16)] = v1
                return carry

            lax.fori_loop(0, msize, row_body, 0, unroll=False)

        def out_copy(obuf_v, task, sem):
            i = task // j_count
            j = task % j_count
            return pltpu.make_async_copy(
                obuf_v,
                out_hbm.at[pl.ds(i * msize, msize),
                           pl.ds(j * TPT * MSIZE, TPT * MSIZE)],
                sem)

        gather(0, tiles[0], gsems[0])

        def loop_body(t, carry):
            for b in range(2):
                local_t = 2 * t + b
                task = task0 + local_t

                @pl.when(local_t + 1 < tasks_per_w)
                def _():
                    gather(local_t + 1, tiles[1 - b], gsems[1 - b])

                gather_wait(tiles[b], gsems[b])

                @pl.when(local_t >= 2)
                def _():
                    out_copy(obufs[b], task - 2, osems[b]).wait()

                transpose(tiles[b])
                repack(obufs[b])
                out_copy(obufs[b], task, osems[b]).start()
            return carry

        lax.fori_loop(0, tasks_per_w // 2, loop_body, 0, unroll=False)

        out_copy(obufs[0], task0 + tasks_per_w - 2, osems[0]).wait()
        out_copy(obufs[1], task0 + tasks_per_w - 1, osems[1]).wait()

    return run(parts_flat, w2d)
